# baseline (speedup 1.0000x reference)
import jax
import jax.numpy as jnp
from jax import lax
from jax.experimental import pallas as pl
from jax.experimental.pallas import tpu as pltpu
from jax.experimental.pallas import tpu_sc as plsc

INPUT_DIM = 512
NUM_NEURONS = 512
BATCH = 1024
TIMESTEPS = 100

_NC = 2
_NS = 16
_ROWS = BATCH // (_NC * _NS)
_NSL = INPUT_DIM // 16
_PLANE = BATCH * NUM_NEURONS
_HALF = _PLANE // 2
_ZC = _HALF // 4
_NCHUNK = TIMESTEPS * _HALF // _ZC
_NSLICE = _ROWS * _NSL
_NXFER = _NSLICE // 128


def _body(x_hbm, out_hbm, x_v, idx3, zstage, ones2, zshared, sem_z, sem_s):
    cid = lax.axis_index("c")
    sid = lax.axis_index("s")
    base = cid * (BATCH // _NC) + sid * _ROWS
    pltpu.sync_copy(x_hbm.at[pl.ds(base, _ROWS)], x_v)

    zero_f = jnp.zeros((16,), jnp.float32)
    one_f = jnp.ones((16,), jnp.float32)
    lane = lax.iota(jnp.int32, 16)

    def _zs(i, _):
        zstage[pl.ds(i * 16, 16)] = zero_f
        return 0

    lax.fori_loop(0, _ZC // _NS // 16, _zs, 0)
    pltpu.sync_copy(zstage, zshared.at[pl.ds(sid * (_ZC // _NS), _ZC // _NS)])
    plsc.subcore_barrier()

    def _zaddr(k):
        t = k // 4
        return t * _PLANE + cid * _HALF + (k % 4) * _ZC

    nz = (_NCHUNK - sid + _NS - 1) // _NS

    def _zfire(i, _):
        k = sid + i * _NS
        pltpu.make_async_copy(zshared, out_hbm.at[pl.ds(_zaddr(k), _ZC)], sem_z).start()
        return 0

    lax.fori_loop(0, nz, _zfire, 0)

    def _st(s, _):
        r = s // _NSL
        c = (s % _NSL) * 16
        xs = x_v[r, pl.ds(c, 16)]
        sig = 1.0 / (1.0 + jnp.exp(-xs))
        st = (sig * jnp.float32(TIMESTEPS - 1)).astype(jnp.int32)
        idx = st * _PLANE + (base + r) * NUM_NEURONS + c + lane
        idx3[s // 128, pl.ds((s % 128) * 16, 16)] = idx
        return 0

    lax.fori_loop(0, _NSLICE, _st, 0)

    def _o2(i, _):
        ones2[pl.ds(i * 16, 16)] = one_f
        return 0

    lax.fori_loop(0, 128, _o2, 0)

    def _zdrain(i, _):
        pltpu.make_async_copy(zshared, out_hbm.at[pl.ds(0, _ZC)], sem_z).wait()
        return 0

    lax.fori_loop(0, nz, _zdrain, 0)
    plsc.subcore_barrier()

    def _sfire(g, _):
        pltpu.make_async_copy(ones2, out_hbm.at[idx3.at[g]], sem_s).start()
        return 0

    lax.fori_loop(0, _NXFER, _sfire, 0)

    def _sdrain(g, _):
        pltpu.make_async_copy(ones2, out_hbm.at[idx3.at[g]], sem_s).wait()
        return 0

    lax.fori_loop(0, _NXFER, _sdrain, 0)


def kernel(continuous_input, timesteps):
    del timesteps
    mesh = plsc.VectorSubcoreMesh(core_axis_name="c", subcore_axis_name="s")
    run = pl.kernel(
        _body,
        out_type=jax.ShapeDtypeStruct((TIMESTEPS * BATCH * NUM_NEURONS,), jnp.float32),
        mesh=mesh,
        scratch_types=[
            pltpu.VMEM((_ROWS, INPUT_DIM), jnp.float32),
            pltpu.VMEM((_NXFER, 2048), jnp.int32),
            pltpu.VMEM((_ZC // _NS,), jnp.float32),
            pltpu.VMEM((2048,), jnp.float32),
            pltpu.VMEM_SHARED((_ZC,), jnp.float32),
            pltpu.SemaphoreType.DMA,
            pltpu.SemaphoreType.DMA,
        ],
        compiler_params=pltpu.CompilerParams(
            use_tc_tiling_on_sc=False, needs_layout_passes=False
        ),
    )
    flat = run(continuous_input)
    return flat.reshape(TIMESTEPS, BATCH, NUM_NEURONS)


# speedup vs baseline: 2.7214x; 2.7214x over previous
import jax
import jax.numpy as jnp
from jax import lax
from jax.experimental import pallas as pl
from jax.experimental.pallas import tpu as pltpu
from jax.experimental.pallas import tpu_sc as plsc

INPUT_DIM = 512
NUM_NEURONS = 512
BATCH = 1024
TIMESTEPS = 100

_NC = 2
_NS = 16
_NW = _NC * _NS
_ROWS = BATCH // _NW
_NSL = INPUT_DIM // 16
_PAIRS = _ROWS // 2


def _body(x_hbm, out_hbm, x_v, buf0, buf1, strow, sem0, sem1):
    wid = lax.axis_index("s") * _NC + lax.axis_index("c")
    base = wid * _ROWS
    pltpu.sync_copy(x_hbm.at[pl.ds(base, _ROWS)], x_v)

    zero_f = jnp.zeros((16,), jnp.float32)
    one_f = jnp.ones((16,), jnp.float32)
    zero_i = jnp.zeros((16,), jnp.int32)
    lane = lax.iota(jnp.int32, 16)

    def _clear(buf, i, _):
        buf[i // _NSL, pl.ds((i % _NSL) * 16, 16)] = zero_f
        return 0

    lax.fori_loop(0, TIMESTEPS * _NSL, lambda i, c: _clear(buf0, i, c), 0)
    lax.fori_loop(0, TIMESTEPS * _NSL, lambda i, c: _clear(buf1, i, c), 0)

    def _zs(j, _):
        strow[0, pl.ds(j * 16, 16)] = zero_i
        strow[1, pl.ds(j * 16, 16)] = zero_i
        return 0

    lax.fori_loop(0, _NSL, _zs, 0)

    def _fill(buf, p, r):
        def _slice(j, _):
            col = lane + j * 16
            old = strow[p, pl.ds(j * 16, 16)]
            plsc.store_scatter(buf, [old, col], zero_f)
            xs = x_v[r, pl.ds(j * 16, 16)]
            sig = 1.0 / (1.0 + jnp.exp(-xs))
            st = (sig * jnp.float32(TIMESTEPS - 1)).astype(jnp.int32)
            plsc.store_scatter(buf, [st, col], one_f)
            strow[p, pl.ds(j * 16, 16)] = st
            return 0

        lax.fori_loop(0, _NSL, _slice, 0)

    def _dma(buf, r, sem):
        return pltpu.make_async_copy(buf, out_hbm.at[:, base + r, :], sem)

    _fill(buf0, 0, 0)
    _dma(buf0, 0, sem0).start()
    _fill(buf1, 1, 1)
    _dma(buf1, 1, sem1).start()

    def _pair(i, _):
        r = 2 * i
        _dma(buf0, r - 2, sem0).wait()
        _fill(buf0, 0, r)
        _dma(buf0, r, sem0).start()
        _dma(buf1, r - 1, sem1).wait()
        _fill(buf1, 1, r + 1)
        _dma(buf1, r + 1, sem1).start()
        return 0

    lax.fori_loop(1, _PAIRS, _pair, 0)
    _dma(buf0, _ROWS - 2, sem0).wait()
    _dma(buf1, _ROWS - 1, sem1).wait()


def kernel(continuous_input, timesteps):
    del timesteps
    mesh = plsc.VectorSubcoreMesh(core_axis_name="c", subcore_axis_name="s")
    run = pl.kernel(
        _body,
        out_type=jax.ShapeDtypeStruct((TIMESTEPS, BATCH, NUM_NEURONS), jnp.float32),
        mesh=mesh,
        scratch_types=[
            pltpu.VMEM((_ROWS, INPUT_DIM), jnp.float32),
            pltpu.VMEM((TIMESTEPS, NUM_NEURONS), jnp.float32),
            pltpu.VMEM((TIMESTEPS, NUM_NEURONS), jnp.float32),
            pltpu.VMEM((2, INPUT_DIM), jnp.int32),
            pltpu.SemaphoreType.DMA,
            pltpu.SemaphoreType.DMA,
        ],
        compiler_params=pltpu.CompilerParams(
            use_tc_tiling_on_sc=False, needs_layout_passes=False
        ),
    )
    return run(continuous_input)
